# baseline (speedup 1.0000x reference)
import functools

import jax
import jax.numpy as jnp
from jax.experimental import pallas as pl

_BETA = 0.25
_K = 512
_D = 64


def _dot(a, b):
    return jnp.dot(a, b, preferred_element_type=jnp.float32)


def _enc1_body(p_ref, w_ref, b_ref, o_ref):
    y = _dot(p_ref[0], w_ref[...]) + b_ref[...]
    o_ref[0] = jnp.maximum(y, 0.0)


def _enc1(patches, w16, b):
    n = patches.shape[0]
    return pl.pallas_call(
        _enc1_body,
        grid=(n,),
        in_specs=[
            pl.BlockSpec((1, 16384, 16), lambda i: (i, 0, 0)),
            pl.BlockSpec((16, 64), lambda i: (0, 0)),
            pl.BlockSpec((1, 64), lambda i: (0, 0)),
        ],
        out_specs=pl.BlockSpec((1, 16384, 64), lambda i: (i, 0, 0)),
        out_shape=jax.ShapeDtypeStruct((n, 16384, 64), jnp.float32),
    )(patches, w16, b)


def _enc2_body(ph_ref, w_ref, b_ref, o_ref):
    acc = jnp.zeros((4096, 64), jnp.float32)
    for di in range(4):
        for dj in range(4):
            t = di * 4 + dj
            win = ph_ref[0, (di % 2) * 2 + (dj % 2),
                         di // 2:di // 2 + 64, dj // 2:dj // 2 + 64, :]
            acc = acc + _dot(win.reshape(4096, 64), w_ref[t])
    o_ref[0] = jnp.maximum(acc + b_ref[...], 0.0)


def _enc2(phases, w, b):
    n = phases.shape[0]
    return pl.pallas_call(
        _enc2_body,
        grid=(n,),
        in_specs=[
            pl.BlockSpec((1, 4, 65, 65, 64), lambda i: (i, 0, 0, 0, 0)),
            pl.BlockSpec((16, 64, 64), lambda i: (0, 0, 0)),
            pl.BlockSpec((1, 64), lambda i: (0, 0)),
        ],
        out_specs=pl.BlockSpec((1, 4096, 64), lambda i: (i, 0, 0)),
        out_shape=jax.ShapeDtypeStruct((n, 4096, 64), jnp.float32),
    )(phases, w, b)


def _vq_body(f_ref, cbt_ref, cbn_ref, cb_ref, idx_ref, q_ref, loss_ref, *,
             scale):
    i = pl.program_id(0)
    flat = f_ref[...]
    scores = cbn_ref[...] - 2.0 * _dot(flat, cbt_ref[...])
    m = jnp.min(scores, axis=1, keepdims=True)
    iota = jax.lax.broadcasted_iota(jnp.int32, scores.shape, 1)
    idx = jnp.min(jnp.where(scores == m, iota, _K), axis=1)
    idx_ref[0] = idx[None, :]
    onehot = (iota == idx[:, None]).astype(jnp.float32)
    q_ref[...] = _dot(onehot, cb_ref[...])
    znorm = jnp.sum(flat * flat, axis=1)
    part = jnp.sum(znorm + m[:, 0], keepdims=True)[None, :] * scale

    @pl.when(i == 0)
    def _():
        loss_ref[...] = jnp.zeros((1, 1), jnp.float32)

    loss_ref[...] += part


def _vq(flat, cbt, cbn, cb):
    tile = 2048
    tiles = flat.shape[0] // tile
    body = functools.partial(
        _vq_body, scale=(1.0 + _BETA) / (flat.shape[0] * 64.0))
    return pl.pallas_call(
        body,
        grid=(tiles,),
        in_specs=[
            pl.BlockSpec((tile, 64), lambda i: (i, 0)),
            pl.BlockSpec((64, _K), lambda i: (0, 0)),
            pl.BlockSpec((1, _K), lambda i: (0, 0)),
            pl.BlockSpec((_K, 64), lambda i: (0, 0)),
        ],
        out_specs=[
            pl.BlockSpec((1, 1, tile), lambda i: (i, 0, 0)),
            pl.BlockSpec((tile, 64), lambda i: (i, 0)),
            pl.BlockSpec((1, 1), lambda i: (0, 0)),
        ],
        out_shape=[
            jax.ShapeDtypeStruct((tiles, 1, tile), jnp.int32),
            jax.ShapeDtypeStruct((tiles * tile, 64), jnp.float32),
            jax.ShapeDtypeStruct((1, 1), jnp.float32),
        ],
    )(flat, cbt, cbn, cb)


_TAPS = {0: (1, 3), 1: (0, 2)}


def _off(p, t):
    return 1 - (t - 1 - p) // 2


def _dect2_body(q_ref, w_ref, b_ref, o_ref):
    for ph in range(2):
        for pw in range(2):
            acc = jnp.zeros((4096, 64), jnp.float32)
            for th in _TAPS[ph]:
                for tw in _TAPS[pw]:
                    oh, ow = _off(ph, th), _off(pw, tw)
                    win = q_ref[0, oh:oh + 64, ow:ow + 64, :]
                    acc = acc + _dot(win.reshape(4096, 64),
                                     w_ref[th * 4 + tw])
            o_ref[0, ph, pw] = jnp.maximum(acc + b_ref[...], 0.0)


def _dect2(q_pad, w, b):
    n = q_pad.shape[0]
    return pl.pallas_call(
        _dect2_body,
        grid=(n,),
        in_specs=[
            pl.BlockSpec((1, 66, 66, 64), lambda i: (i, 0, 0, 0)),
            pl.BlockSpec((16, 64, 64), lambda i: (0, 0, 0)),
            pl.BlockSpec((1, 64), lambda i: (0, 0)),
        ],
        out_specs=pl.BlockSpec((1, 2, 2, 4096, 64), lambda i: (i, 0, 0, 0, 0)),
        out_shape=jax.ShapeDtypeStruct((n, 2, 2, 4096, 64), jnp.float32),
    )(q_pad, w, b)


def _dect1_body(h_ref, w_ref, b_ref, o_ref):
    t = _dot(h_ref[0].reshape(136 * 136, 64), w_ref[...])
    t = t.reshape(136, 136, 16)
    for ph in range(2):
        for pw in range(2):
            acc = jnp.zeros((128, 128), jnp.float32)
            for th in _TAPS[ph]:
                for tw in _TAPS[pw]:
                    oh, ow = _off(ph, th), _off(pw, tw)
                    acc = acc + t[oh:oh + 128, ow:ow + 128, th * 4 + tw]
            o_ref[0, ph, pw] = jax.nn.sigmoid(acc + b_ref[0, 0])


def _dect1(h_pad, w16, b):
    n = h_pad.shape[0]
    return pl.pallas_call(
        _dect1_body,
        grid=(n,),
        in_specs=[
            pl.BlockSpec((1, 136, 136, 64), lambda i: (i, 0, 0, 0)),
            pl.BlockSpec((64, 16), lambda i: (0, 0)),
            pl.BlockSpec((1, 1), lambda i: (0, 0)),
        ],
        out_specs=pl.BlockSpec((1, 2, 2, 128, 128), lambda i: (i, 0, 0, 0, 0)),
        out_shape=jax.ShapeDtypeStruct((n, 2, 2, 128, 128), jnp.float32),
    )(h_pad, w16, b)


@jax.jit
def kernel(x, enc_w1, enc_b1, enc_w2, enc_b2, dec_w2, dec_b2, dec_w1, dec_b1,
           codebook):
    n = x.shape[0]

    xp = jnp.pad(x[:, 0], ((0, 0), (1, 1), (1, 1)))
    cols = []
    for di in range(4):
        for dj in range(4):
            cols.append(jax.lax.slice(
                xp, (0, di, dj), (n, di + 255, dj + 255), (1, 2, 2)))
    patches = jnp.stack(cols, axis=-1).reshape(n, 16384, 16)
    w1 = enc_w1.reshape(64, 16).T
    h = _enc1(patches, w1, enc_b1[None, :])

    h = h.reshape(n, 128, 128, 64)
    hp = jnp.pad(h, ((0, 0), (1, 1), (1, 1), (0, 0)))
    hp = hp.reshape(n, 65, 2, 65, 2, 64).transpose(0, 2, 4, 1, 3, 5)
    hp = hp.reshape(n, 4, 65, 65, 64)
    w2 = enc_w2.reshape(64, 64, 16).transpose(2, 1, 0)
    lat = _enc2(hp, w2, enc_b2[None, :])

    flat = lat.reshape(n * 4096, 64)
    cbt = codebook.T
    cbn = jnp.sum(codebook * codebook, axis=1)[None, :]
    idx, quant, loss = _vq(flat, cbt, cbn, codebook)
    indices = idx.reshape(n * 4096)[:, None]

    q = quant.reshape(n, 64, 64, 64)
    qp = jnp.pad(q, ((0, 0), (1, 1), (1, 1), (0, 0)))
    wd2 = dec_w2.reshape(64, 64, 16).transpose(2, 0, 1)
    hd = _dect2(qp, wd2, dec_b2[None, :])
    hd = hd.reshape(n, 2, 2, 64, 64, 64).transpose(0, 3, 1, 4, 2, 5)
    hd = hd.reshape(n, 128, 128, 64)

    hdp = jnp.pad(hd, ((0, 0), (1, 5), (1, 5), (0, 0)))
    wd1 = dec_w1.reshape(64, 16)
    xr = _dect1(hdp, wd1, dec_b1[None, :])
    xr = xr.transpose(0, 3, 1, 4, 2).reshape(n, 1, 256, 256)

    return (loss[0, 0], indices, xr)


# speedup vs baseline: 3.1931x; 3.1931x over previous
import jax
import jax.numpy as jnp
from jax.experimental import pallas as pl
from jax.experimental.pallas import tpu as pltpu

_BETA = 0.25
_K = 512

_TAPS = {0: (1, 3), 1: (0, 2)}


def _off(p, t):
    return 1 - (t - 1 - p) // 2


def _dot(a, b):
    return jnp.dot(a, b, preferred_element_type=jnp.float32)


def _fused_body(x16_ref, w1_ref, b1_ref, w2_ref, b2_ref, cbt_ref, cbn_ref,
                cb_ref, wd2_ref, bd2_ref, wd1_ref, bd1_ref,
                idx_ref, xr_ref, loss_ref,
                hph_ref, qpad_ref, hdp_ref, *, scale):
    f32 = jnp.float32

    for qa in range(2):
        for qb in range(2):
            acc = jnp.zeros((64, 64, 64), f32)
            for di in range(4):
                for dj in range(4):
                    r, c = 2 * qa + di, 2 * qb + dj
                    win = x16_ref[0, (r % 4) * 4 + (c % 4),
                                  r // 4:r // 4 + 64, c // 4:c // 4 + 64]
                    acc = acc + win[:, :, None] * w1_ref[di * 4 + dj][None,
                                                                      None, :]
            plane = jnp.maximum(acc + b1_ref[0][None, None, :], 0.0)
            a, b = (qa + 1) % 2, (qb + 1) % 2
            hph_ref[a, b, 64 * a:64 * a + 1, :, :] = jnp.zeros((1, 65, 64),
                                                               f32)
            hph_ref[a, b, :, 64 * b:64 * b + 1, :] = jnp.zeros((65, 1, 64),
                                                               f32)
            hph_ref[a, b, qa:qa + 64, qb:qb + 64, :] = plane

    acc = jnp.zeros((4096, 64), f32)
    for di in range(4):
        for dj in range(4):
            win = hph_ref[di % 2, dj % 2,
                          di // 2:di // 2 + 64, dj // 2:dj // 2 + 64, :]
            acc = acc + _dot(win.reshape(4096, 64), w2_ref[di * 4 + dj])
    flat = jnp.maximum(acc + b2_ref[...], 0.0)

    scores = cbn_ref[...] - 2.0 * _dot(flat, cbt_ref[...])
    m = jnp.min(scores, axis=1, keepdims=True)
    iota = jax.lax.broadcasted_iota(jnp.int32, scores.shape, 1)
    idx = jnp.min(jnp.where(scores == m, iota, _K), axis=1)
    idx_ref[0] = idx[None, :]
    onehot = (iota == idx[:, None]).astype(f32)
    quant = _dot(onehot, cb_ref[...])
    part = (jnp.sum(flat * flat) + jnp.sum(m)) * scale
    loss_ref[0] = part.reshape(1, 1)

    qpad_ref[0:1, :, :] = jnp.zeros((1, 66, 64), f32)
    qpad_ref[65:66, :, :] = jnp.zeros((1, 66, 64), f32)
    qpad_ref[:, 0:1, :] = jnp.zeros((66, 1, 64), f32)
    qpad_ref[:, 65:66, :] = jnp.zeros((66, 1, 64), f32)
    qpad_ref[1:65, 1:65, :] = quant.reshape(64, 64, 64)

    for ph in range(2):
        for pw in range(2):
            acc = jnp.zeros((4096, 64), f32)
            for th in _TAPS[ph]:
                for tw in _TAPS[pw]:
                    oh, ow = _off(ph, th), _off(pw, tw)
                    win = qpad_ref[oh:oh + 64, ow:ow + 64, :]
                    acc = acc + _dot(win.reshape(4096, 64),
                                     wd2_ref[th * 4 + tw])
            plane = jnp.maximum(acc + bd2_ref[...], 0.0)
            hdp_ref[ph, pw, 0:1, :, :] = jnp.zeros((1, 72, 64), f32)
            hdp_ref[ph, pw, 65:66, :, :] = jnp.zeros((1, 72, 64), f32)
            hdp_ref[ph, pw, :, 0:1, :] = jnp.zeros((66, 1, 64), f32)
            hdp_ref[ph, pw, :, 65:72, :] = jnp.zeros((66, 7, 64), f32)
            hdp_ref[ph, pw, 1:65, 1:65, :] = plane.reshape(64, 64, 64)

    tpl = {}
    for p in range(2):
        for q in range(2):
            t = _dot(hdp_ref[p, q].reshape(66 * 72, 64), wd1_ref[...])
            tpl[(p, q)] = t.reshape(66, 72, 16)
    for po_h in range(2):
        for e_h in range(2):
            for po_w in range(2):
                for e_w in range(2):
                    acc2 = jnp.zeros((64, 64), f32)
                    for th in _TAPS[po_h]:
                        for tw in _TAPS[po_w]:
                            g_h = e_h + (po_h - th + 1) // 2
                            g_w = e_w + (po_w - tw + 1) // 2
                            t = tpl[(g_h % 2, g_w % 2)]
                            oh, ow = g_h // 2 + 1, g_w // 2 + 1
                            acc2 = acc2 + t[oh:oh + 64, ow:ow + 64,
                                            th * 4 + tw]
                    xr_ref[0, po_h, e_h, po_w, e_w] = jax.nn.sigmoid(
                        acc2 + bd1_ref[0, 0])


def _fused(x16, w1, b1, w2, b2, cbt, cbn, cb, wd2, bd2, wd1, bd1, scale):
    import functools
    n = x16.shape[0]
    body = functools.partial(_fused_body, scale=scale)
    return pl.pallas_call(
        body,
        grid=(n,),
        in_specs=[
            pl.BlockSpec((1, 16, 65, 65), lambda i: (i, 0, 0, 0)),
            pl.BlockSpec((16, 64), lambda i: (0, 0)),
            pl.BlockSpec((1, 64), lambda i: (0, 0)),
            pl.BlockSpec((16, 64, 64), lambda i: (0, 0, 0)),
            pl.BlockSpec((1, 64), lambda i: (0, 0)),
            pl.BlockSpec((64, _K), lambda i: (0, 0)),
            pl.BlockSpec((1, _K), lambda i: (0, 0)),
            pl.BlockSpec((_K, 64), lambda i: (0, 0)),
            pl.BlockSpec((16, 64, 64), lambda i: (0, 0, 0)),
            pl.BlockSpec((1, 64), lambda i: (0, 0)),
            pl.BlockSpec((64, 16), lambda i: (0, 0)),
            pl.BlockSpec((1, 1), lambda i: (0, 0)),
        ],
        out_specs=[
            pl.BlockSpec((1, 1, 4096), lambda i: (i, 0, 0)),
            pl.BlockSpec((1, 2, 2, 2, 2, 64, 64),
                         lambda i: (i, 0, 0, 0, 0, 0, 0)),
            pl.BlockSpec((1, 1, 1), lambda i: (i, 0, 0)),
        ],
        out_shape=[
            jax.ShapeDtypeStruct((n, 1, 4096), jnp.int32),
            jax.ShapeDtypeStruct((n, 2, 2, 2, 2, 64, 64), jnp.float32),
            jax.ShapeDtypeStruct((n, 1, 1), jnp.float32),
        ],
        scratch_shapes=[
            pltpu.VMEM((2, 2, 65, 65, 64), jnp.float32),
            pltpu.VMEM((66, 66, 64), jnp.float32),
            pltpu.VMEM((2, 2, 66, 72, 64), jnp.float32),
        ],
        compiler_params=pltpu.CompilerParams(
            dimension_semantics=("arbitrary",)),
    )(x16, w1, b1, w2, b2, cbt, cbn, cb, wd2, bd2, wd1, bd1)


@jax.jit
def kernel(x, enc_w1, enc_b1, enc_w2, enc_b2, dec_w2, dec_b2, dec_w1, dec_b1,
           codebook):
    n = x.shape[0]

    xp = jnp.pad(x[:, 0], ((0, 0), (1, 3), (1, 3)))
    x16 = xp.reshape(n, 65, 4, 65, 4).transpose(0, 2, 4, 1, 3)
    x16 = x16.reshape(n, 16, 65, 65)

    w1 = enc_w1.reshape(64, 16).T
    w2 = enc_w2.reshape(64, 64, 16).transpose(2, 1, 0)
    wd2 = dec_w2.reshape(64, 64, 16).transpose(2, 0, 1)
    wd1 = dec_w1.reshape(64, 16)
    cbt = codebook.T
    cbn = jnp.sum(codebook * codebook, axis=1)[None, :]
    scale = (1.0 + _BETA) / (n * 4096 * 64.0)

    idx, xr, loss_parts = _fused(
        x16, w1, enc_b1[None, :], w2, enc_b2[None, :], cbt, cbn, codebook,
        wd2, dec_b2[None, :], wd1, dec_b1[None, :], scale)

    indices = idx.reshape(n * 4096)[:, None]
    x_recon = xr.transpose(0, 5, 2, 1, 6, 4, 3).reshape(n, 1, 256, 256)
    return (jnp.sum(loss_parts), indices, x_recon)


# speedup vs baseline: 3.1948x; 1.0005x over previous
import jax
import jax.numpy as jnp
from jax.experimental import pallas as pl
from jax.experimental.pallas import tpu as pltpu

_BETA = 0.25
_K = 512

_TAPS = {0: (1, 3), 1: (0, 2)}


def _off(p, t):
    return 1 - (t - 1 - p) // 2


def _dot(a, b):
    return jnp.dot(a, b, preferred_element_type=jnp.float32)


def _fused_body(x16_ref, w1_ref, b1_ref, w2_ref, b2_ref, cbt_ref, cbn_ref,
                cb_ref, wd2_ref, bd2_ref, wd1_ref, bd1_ref,
                idx_ref, xr_ref, loss_ref,
                hph_ref, qpad_ref, hdp_ref, *, scale):
    f32 = jnp.float32

    for qa in range(2):
        for qb in range(2):
            acc = jnp.zeros((64, 64, 64), f32)
            for di in range(4):
                for dj in range(4):
                    r, c = 2 * qa + di, 2 * qb + dj
                    win = x16_ref[0, (r % 4) * 4 + (c % 4),
                                  r // 4:r // 4 + 64, c // 4:c // 4 + 64]
                    acc = acc + win[:, :, None] * w1_ref[di * 4 + dj][None,
                                                                      None, :]
            plane = jnp.maximum(acc + b1_ref[0][None, None, :], 0.0)
            a, b = (qa + 1) % 2, (qb + 1) % 2
            hph_ref[a, b, 64 * a:64 * a + 1, :, :] = jnp.zeros((1, 65, 64),
                                                               f32)
            hph_ref[a, b, :, 64 * b:64 * b + 1, :] = jnp.zeros((65, 1, 64),
                                                               f32)
            hph_ref[a, b, qa:qa + 64, qb:qb + 64, :] = plane

    acc = jnp.zeros((4096, 64), f32)
    for di in range(4):
        for dj in range(4):
            win = hph_ref[di % 2, dj % 2,
                          di // 2:di // 2 + 64, dj // 2:dj // 2 + 64, :]
            acc = acc + _dot(win.reshape(4096, 64), w2_ref[di * 4 + dj])
    flat = jnp.maximum(acc + b2_ref[...], 0.0)

    scores = cbn_ref[...] - 2.0 * _dot(flat, cbt_ref[...])
    m = jnp.min(scores, axis=1, keepdims=True)
    iota = jax.lax.broadcasted_iota(jnp.int32, scores.shape, 1)
    idx = jnp.min(jnp.where(scores == m, iota, _K), axis=1)
    idx_ref[0] = idx[None, :]
    onehot = (iota == idx[:, None]).astype(f32)
    quant = _dot(onehot, cb_ref[...])
    part = (jnp.sum(flat * flat) + jnp.sum(m)) * scale
    loss_ref[0] = part.reshape(1, 1)

    qpad_ref[0:1, :, :] = jnp.zeros((1, 66, 64), f32)
    qpad_ref[65:66, :, :] = jnp.zeros((1, 66, 64), f32)
    qpad_ref[:, 0:1, :] = jnp.zeros((66, 1, 64), f32)
    qpad_ref[:, 65:66, :] = jnp.zeros((66, 1, 64), f32)
    qpad_ref[1:65, 1:65, :] = quant.reshape(64, 64, 64)

    for ph in range(2):
        for pw in range(2):
            acc = jnp.zeros((4096, 64), f32)
            for th in _TAPS[ph]:
                for tw in _TAPS[pw]:
                    oh, ow = _off(ph, th), _off(pw, tw)
                    win = qpad_ref[oh:oh + 64, ow:ow + 64, :]
                    acc = acc + _dot(win.reshape(4096, 64),
                                     wd2_ref[th * 4 + tw])
            plane = jnp.maximum(acc + bd2_ref[...], 0.0)
            hdp_ref[ph, pw, 0:1, :, :] = jnp.zeros((1, 72, 64), f32)
            hdp_ref[ph, pw, 65:66, :, :] = jnp.zeros((1, 72, 64), f32)
            hdp_ref[ph, pw, :, 0:1, :] = jnp.zeros((66, 1, 64), f32)
            hdp_ref[ph, pw, :, 65:72, :] = jnp.zeros((66, 7, 64), f32)
            hdp_ref[ph, pw, 1:65, 1:65, :] = plane.reshape(64, 64, 64)

    tpl = {}
    for p in range(2):
        for q in range(2):
            t = _dot(hdp_ref[p, q].reshape(66 * 72, 64), wd1_ref[...])
            tpl[(p, q)] = t.reshape(66, 72, 16)
    for po_h in range(2):
        for e_h in range(2):
            for po_w in range(2):
                for e_w in range(2):
                    acc2 = jnp.zeros((64, 64), f32)
                    for th in _TAPS[po_h]:
                        for tw in _TAPS[po_w]:
                            g_h = e_h + (po_h - th + 1) // 2
                            g_w = e_w + (po_w - tw + 1) // 2
                            t = tpl[(g_h % 2, g_w % 2)]
                            oh, ow = g_h // 2 + 1, g_w // 2 + 1
                            acc2 = acc2 + t[oh:oh + 64, ow:ow + 64,
                                            th * 4 + tw]
                    xr_ref[0, po_h, e_h, po_w, e_w] = jax.nn.sigmoid(
                        acc2 + bd1_ref[0, 0])


def _fused(x16, w1, b1, w2, b2, cbt, cbn, cb, wd2, bd2, wd1, bd1, scale):
    import functools
    n = x16.shape[0]
    body = functools.partial(_fused_body, scale=scale)
    return pl.pallas_call(
        body,
        grid=(n,),
        in_specs=[
            pl.BlockSpec((1, 16, 65, 65), lambda i: (i, 0, 0, 0)),
            pl.BlockSpec((16, 64), lambda i: (0, 0)),
            pl.BlockSpec((1, 64), lambda i: (0, 0)),
            pl.BlockSpec((16, 64, 64), lambda i: (0, 0, 0)),
            pl.BlockSpec((1, 64), lambda i: (0, 0)),
            pl.BlockSpec((64, _K), lambda i: (0, 0)),
            pl.BlockSpec((1, _K), lambda i: (0, 0)),
            pl.BlockSpec((_K, 64), lambda i: (0, 0)),
            pl.BlockSpec((16, 64, 64), lambda i: (0, 0, 0)),
            pl.BlockSpec((1, 64), lambda i: (0, 0)),
            pl.BlockSpec((64, 16), lambda i: (0, 0)),
            pl.BlockSpec((1, 1), lambda i: (0, 0)),
        ],
        out_specs=[
            pl.BlockSpec((1, 1, 4096), lambda i: (i, 0, 0)),
            pl.BlockSpec((1, 2, 2, 2, 2, 64, 64),
                         lambda i: (i, 0, 0, 0, 0, 0, 0)),
            pl.BlockSpec((1, 1, 1), lambda i: (i, 0, 0)),
        ],
        out_shape=[
            jax.ShapeDtypeStruct((n, 1, 4096), jnp.int32),
            jax.ShapeDtypeStruct((n, 2, 2, 2, 2, 64, 64), jnp.float32),
            jax.ShapeDtypeStruct((n, 1, 1), jnp.float32),
        ],
        scratch_shapes=[
            pltpu.VMEM((2, 2, 65, 65, 64), jnp.float32),
            pltpu.VMEM((66, 66, 64), jnp.float32),
            pltpu.VMEM((2, 2, 66, 72, 64), jnp.float32),
        ],
        compiler_params=pltpu.CompilerParams(
            dimension_semantics=("parallel",)),
    )(x16, w1, b1, w2, b2, cbt, cbn, cb, wd2, bd2, wd1, bd1)


@jax.jit
def kernel(x, enc_w1, enc_b1, enc_w2, enc_b2, dec_w2, dec_b2, dec_w1, dec_b1,
           codebook):
    n = x.shape[0]

    xp = jnp.pad(x[:, 0], ((0, 0), (1, 3), (1, 3)))
    x16 = xp.reshape(n, 65, 4, 65, 4).transpose(0, 2, 4, 1, 3)
    x16 = x16.reshape(n, 16, 65, 65)

    w1 = enc_w1.reshape(64, 16).T
    w2 = enc_w2.reshape(64, 64, 16).transpose(2, 1, 0)
    wd2 = dec_w2.reshape(64, 64, 16).transpose(2, 0, 1)
    wd1 = dec_w1.reshape(64, 16)
    cbt = codebook.T
    cbn = jnp.sum(codebook * codebook, axis=1)[None, :]
    scale = (1.0 + _BETA) / (n * 4096 * 64.0)

    idx, xr, loss_parts = _fused(
        x16, w1, enc_b1[None, :], w2, enc_b2[None, :], cbt, cbn, codebook,
        wd2, dec_b2[None, :], wd1, dec_b1[None, :], scale)

    indices = idx.reshape(n * 4096)[:, None]
    x_recon = xr.transpose(0, 5, 2, 1, 6, 4, 3).reshape(n, 1, 256, 256)
    return (jnp.sum(loss_parts), indices, x_recon)
